# baseline (speedup 1.0000x reference)
import math

import jax
import jax.numpy as jnp
from jax.experimental import pallas as pl
from jax.experimental.pallas import tpu as pltpu

_GRID = 192
_SCALE = 0.25
_IMG = 768.0
_OUT = 7
_SR = 2
_NS = _OUT * _SR
_K = 64
_POS_D = 720


def _make_body(h, w, off, bc):
    def body(bT_ref, f_ref, o_ref, rw_ref, cw_ref):
        pid = pl.program_id(0)

        @pl.when(pid == 0)
        def _build_weights():
            bT = bT_ref[...]
            x1 = bT[0:1, :] * _SCALE
            y1 = bT[1:2, :] * _SCALE
            x2 = bT[2:3, :] * _SCALE
            y2 = bT[3:4, :] * _SCALE
            roi_w = jnp.maximum(x2 - x1, 1.0)
            roi_h = jnp.maximum(y2 - y1, 1.0)
            g = (jax.lax.broadcasted_iota(jnp.int32, (_NS, _K), 0)
                 .astype(jnp.float32) + 0.5) / (2.0 * _OUT)
            xs = x1 + g * roi_w
            ys = y1 + g * roi_h

            def grid_tent(s):
                valid = ((s > -1.0) & (s < float(_GRID))).astype(jnp.float32)
                sc = jnp.clip(s, 0.0, float(_GRID - 1))
                i = jax.lax.broadcasted_iota(jnp.int32, (_GRID, _NS, _K),
                                             0).astype(jnp.float32)
                t = jnp.maximum(0.0, 1.0 - jnp.abs(sc[None] - i)) * valid[None]
                return jnp.sum(t, axis=1) * (1.0 / _NS)

            ay = grid_tent(ys)
            ax = grid_tent(xs)

            def up_tent(n):
                i = jax.lax.broadcasted_iota(jnp.int32, (n, _GRID),
                                             1).astype(jnp.float32)
                u = jnp.clip((i + 0.5) * (n / float(_GRID)) - 0.5, 0.0,
                             float(n - 1))
                r = jax.lax.broadcasted_iota(jnp.int32, (n, _GRID),
                                             0).astype(jnp.float32)
                return jnp.maximum(0.0, 1.0 - jnp.abs(u - r))

            rw_ref[...] = jnp.dot(up_tent(h), ay,
                                  preferred_element_type=jnp.float32)
            cw_ref[...] = jnp.dot(up_tent(w), ax,
                                  preferred_element_type=jnp.float32)

        x = f_ref[...].reshape(bc * h, w)
        b = jnp.dot(x, cw_ref[...], preferred_element_type=jnp.float32)
        pooled = jnp.sum(b.reshape(bc, h, _K) * rw_ref[...][None, :, :],
                         axis=1)

        c = off + pid * bc + jax.lax.broadcasted_iota(jnp.int32, (bc, 1), 0)
        blk = c // _POS_D
        j = c - blk * _POS_D
        expo = (2.0 * (j // 2).astype(jnp.float32)) / float(_POS_D)
        inv_t = jnp.exp(-math.log(10000.0) * expo)
        bT = bT_ref[...]
        nx1 = bT[0:1, :] / _IMG
        ny1 = bT[1:2, :] / _IMG
        nw = bT[2:3, :] / _IMG - nx1
        nh = bT[3:4, :] / _IMG - ny1
        cx = nx1 + nw * 0.5
        cy = ny1 + nh * 0.5
        v = jnp.where(blk == 0, cy,
                      jnp.where(blk == 1, cx,
                                jnp.where(blk == 2, nw, nh)))
        ang = v * (2.0 * math.pi) * inv_t
        pe = jnp.where(j % 2 == 0, jnp.sin(ang), jnp.cos(ang))

        o_ref[...] = pooled + pe

    return body


def _level_call(feat, boxes_t, off, bc):
    _, c, h, w = feat.shape
    body = _make_body(h, w, off, bc)
    return pl.pallas_call(
        body,
        out_shape=jax.ShapeDtypeStruct((c, _K), jnp.float32),
        grid=(c // bc,),
        in_specs=[
            pl.BlockSpec((4, _K), lambda i: (0, 0)),
            pl.BlockSpec((1, bc, h, w), lambda i: (0, i, 0, 0)),
        ],
        out_specs=pl.BlockSpec((bc, _K), lambda i: (i, 0)),
        scratch_shapes=[
            pltpu.VMEM((h, _K), jnp.float32),
            pltpu.VMEM((w, _K), jnp.float32),
        ],
        compiler_params=pltpu.CompilerParams(
            dimension_semantics=("arbitrary",)),
        name=f"roi_level_{h}",
    )(boxes_t, feat)


def kernel(feat0, feat1, feat2, feat3, boxes):
    boxes_t = jnp.transpose(boxes, (1, 0))
    outs = []
    off = 0
    for feat, bc in ((feat0, 16), (feat1, 32), (feat2, 64), (feat3, 128)):
        outs.append(_level_call(feat, boxes_t, off, bc))
        off += feat.shape[1]
    full = jnp.concatenate(outs, axis=0)
    return jnp.transpose(full, (1, 0))[None]


# speedup vs baseline: 1.1106x; 1.1106x over previous
import math

import jax
import jax.numpy as jnp
from jax.experimental import pallas as pl
from jax.experimental.pallas import tpu as pltpu

_GRID = 192
_SCALE = 0.25
_IMG = 768.0
_OUT = 7
_SR = 2
_NS = _OUT * _SR
_K = 64
_POS_D = 720
_HS = (192, 96, 48, 24)
_CS = (192, 384, 768, 1536)


def _weights_body(bT_ref, *out_refs):
    bT = bT_ref[...]
    x1 = bT[0:1, :] * _SCALE
    y1 = bT[1:2, :] * _SCALE
    x2 = bT[2:3, :] * _SCALE
    y2 = bT[3:4, :] * _SCALE
    roi_w = jnp.maximum(x2 - x1, 1.0)
    roi_h = jnp.maximum(y2 - y1, 1.0)
    g = (jax.lax.broadcasted_iota(jnp.int32, (_NS, _K), 0)
         .astype(jnp.float32) + 0.5) / (2.0 * _OUT)
    xs = x1 + g * roi_w
    ys = y1 + g * roi_h

    def grid_tent(s):
        valid = ((s > -1.0) & (s < float(_GRID))).astype(jnp.float32)
        sc = jnp.clip(s, 0.0, float(_GRID - 1))
        i = jax.lax.broadcasted_iota(jnp.int32, (_GRID, _NS, _K),
                                     0).astype(jnp.float32)
        t = jnp.maximum(0.0, 1.0 - jnp.abs(sc[None] - i)) * valid[None]
        return jnp.sum(t, axis=1) * (1.0 / _NS)

    ay = grid_tent(ys)
    ax = grid_tent(xs)

    def up_tent(n):
        i = jax.lax.broadcasted_iota(jnp.int32, (n, _GRID),
                                     1).astype(jnp.float32)
        u = jnp.clip((i + 0.5) * (n / float(_GRID)) - 0.5, 0.0, float(n - 1))
        r = jax.lax.broadcasted_iota(jnp.int32, (n, _GRID),
                                     0).astype(jnp.float32)
        return jnp.maximum(0.0, 1.0 - jnp.abs(u - r))

    for lvl, h in enumerate(_HS):
        t = up_tent(h)
        out_refs[2 * lvl][...] = jnp.dot(t, ay,
                                         preferred_element_type=jnp.float32)
        out_refs[2 * lvl + 1][...] = jnp.dot(
            t, ax, preferred_element_type=jnp.float32)

    nx1 = bT[0:1, :] / _IMG
    ny1 = bT[1:2, :] / _IMG
    nw = bT[2:3, :] / _IMG - nx1
    nh = bT[3:4, :] / _IMG - ny1
    cx = nx1 + nw * 0.5
    cy = ny1 + nh * 0.5
    off = 0
    for lvl, cl in enumerate(_CS):
        c = off + jax.lax.broadcasted_iota(jnp.int32, (cl, 1), 0)
        blk = c // _POS_D
        j = c - blk * _POS_D
        expo = (2.0 * (j // 2).astype(jnp.float32)) / float(_POS_D)
        inv_t = jnp.exp(-math.log(10000.0) * expo)
        v = jnp.where(blk == 0, cy,
                      jnp.where(blk == 1, cx,
                                jnp.where(blk == 2, nw, nh)))
        ang = v * (2.0 * math.pi) * inv_t
        out_refs[8 + lvl][...] = jnp.where(j % 2 == 0, jnp.sin(ang),
                                           jnp.cos(ang))
        off += cl


def _build_weights(boxes_t):
    shapes = []
    for h in _HS:
        shapes += [jax.ShapeDtypeStruct((h, _K), jnp.float32)] * 2
    for cl in _CS:
        shapes.append(jax.ShapeDtypeStruct((cl, _K), jnp.float32))
    return pl.pallas_call(
        _weights_body,
        out_shape=shapes,
        name="roi_weights",
    )(boxes_t)


def _make_level_body(h, w, bc):
    def body(rw_ref, cw_ref, pe_ref, f_ref, o_ref):
        x = f_ref[...].reshape(bc * h, w)
        b = jnp.dot(x, cw_ref[...], preferred_element_type=jnp.float32)
        pooled = jnp.sum(b.reshape(bc, h, _K) * rw_ref[...][None, :, :],
                         axis=1)
        o_ref[...] = pooled + pe_ref[...]
    return body


def _level_call(feat, rw, cw, pe, bc):
    _, c, h, w = feat.shape
    body = _make_level_body(h, w, bc)
    return pl.pallas_call(
        body,
        out_shape=jax.ShapeDtypeStruct((c, _K), jnp.float32),
        grid=(c // bc,),
        in_specs=[
            pl.BlockSpec((h, _K), lambda i: (0, 0)),
            pl.BlockSpec((w, _K), lambda i: (0, 0)),
            pl.BlockSpec((bc, _K), lambda i: (i, 0)),
            pl.BlockSpec((1, bc, h, w), lambda i: (0, i, 0, 0)),
        ],
        out_specs=pl.BlockSpec((bc, _K), lambda i: (i, 0)),
        compiler_params=pltpu.CompilerParams(
            dimension_semantics=("arbitrary",)),
        name=f"roi_level_{h}",
    )(rw, cw, pe, feat)


def kernel(feat0, feat1, feat2, feat3, boxes):
    boxes_t = jnp.transpose(boxes, (1, 0))
    ws = _build_weights(boxes_t)
    outs = []
    for lvl, (feat, bc) in enumerate(
            ((feat0, 32), (feat1, 64), (feat2, 128), (feat3, 256))):
        outs.append(_level_call(feat, ws[2 * lvl], ws[2 * lvl + 1],
                                ws[8 + lvl], bc))
    full = jnp.concatenate(outs, axis=0)
    return jnp.transpose(full, (1, 0))[None]


# speedup vs baseline: 1.5235x; 1.3718x over previous
import math

import jax
import jax.numpy as jnp
from jax.experimental import pallas as pl
from jax.experimental.pallas import tpu as pltpu

_GRID = 192
_SCALE = 0.25
_IMG = 768.0
_OUT = 7
_SR = 2
_NS = _OUT * _SR
_K = 64
_POS_D = 720

_BC0, _BC1, _BC2, _BC3 = 16, 64, 256, 256
_N0, _N1, _N2, _N3 = 192 // _BC0, 384 // _BC1, 768 // _BC2, 1536 // _BC3
_S1 = _N0
_S2 = _S1 + _N1
_S3 = _S2 + _N2
_STEPS = _S3 + _N3


def _up_tent(n, rows, row_iota_dim, col_iota_dim, wdiv=None):
    i = jax.lax.broadcasted_iota(jnp.int32, (rows, _GRID),
                                 col_iota_dim).astype(jnp.float32)
    u = jnp.clip((i + 0.5) * (n / float(_GRID)) - 0.5, 0.0, float(n - 1))
    p = jax.lax.broadcasted_iota(jnp.int32, (rows, _GRID), row_iota_dim)
    if wdiv is not None:
        p = p // wdiv
    r = p.astype(jnp.float32)
    return jnp.maximum(0.0, 1.0 - jnp.abs(u - r))


def _up_tent_mod(n, w, rows):
    i = jax.lax.broadcasted_iota(jnp.int32, (rows, _GRID),
                                 1).astype(jnp.float32)
    u = jnp.clip((i + 0.5) * (n / float(_GRID)) - 0.5, 0.0, float(n - 1))
    p = jax.lax.broadcasted_iota(jnp.int32, (rows, _GRID), 0)
    q = (p - (p // w) * w).astype(jnp.float32)
    return jnp.maximum(0.0, 1.0 - jnp.abs(u - q))


def _body(bT_ref, f0_ref, f1_ref, f2_ref, f3_ref, o_ref,
          ay_ref, ax_ref, rw0_ref, cw0_ref, rw1_ref, cw1_ref,
          w2_ref, w3_ref):
    i = pl.program_id(0)
    bT = bT_ref[...]

    def pos_embed(base, bc):
        c = base + jax.lax.broadcasted_iota(jnp.int32, (bc, 1), 0)
        blk = c // _POS_D
        j = c - blk * _POS_D
        expo = (2.0 * (j // 2).astype(jnp.float32)) / float(_POS_D)
        inv_t = jnp.exp(-math.log(10000.0) * expo)
        nx1 = bT[0:1, :] / _IMG
        ny1 = bT[1:2, :] / _IMG
        nw = bT[2:3, :] / _IMG - nx1
        nh = bT[3:4, :] / _IMG - ny1
        v = jnp.where(blk == 0, ny1 + nh * 0.5,
                      jnp.where(blk == 1, nx1 + nw * 0.5,
                                jnp.where(blk == 2, nw, nh)))
        ang = v * (2.0 * math.pi) * inv_t
        return jnp.where(j % 2 == 0, jnp.sin(ang), jnp.cos(ang))

    @pl.when(i == 0)
    def _init():
        x1 = bT[0:1, :] * _SCALE
        y1 = bT[1:2, :] * _SCALE
        x2 = bT[2:3, :] * _SCALE
        y2 = bT[3:4, :] * _SCALE
        roi_w = jnp.maximum(x2 - x1, 1.0)
        roi_h = jnp.maximum(y2 - y1, 1.0)
        g = (jax.lax.broadcasted_iota(jnp.int32, (_NS, _K), 0)
             .astype(jnp.float32) + 0.5) / (2.0 * _OUT)
        xs = x1 + g * roi_w
        ys = y1 + g * roi_h

        def grid_tent(s):
            valid = ((s > -1.0) & (s < float(_GRID))).astype(jnp.float32)
            sc = jnp.clip(s, 0.0, float(_GRID - 1))
            gi = jax.lax.broadcasted_iota(jnp.int32, (_GRID, _NS, _K),
                                          0).astype(jnp.float32)
            t = jnp.maximum(0.0, 1.0 - jnp.abs(sc[None] - gi)) * valid[None]
            return jnp.sum(t, axis=1) * (1.0 / _NS)

        ay_ref[...] = grid_tent(ys)
        ax_ref[...] = grid_tent(xs)
        t0 = _up_tent(192, 192, 0, 1)
        rw0_ref[...] = jnp.dot(t0, ay_ref[...],
                               preferred_element_type=jnp.float32)
        cw0_ref[...] = jnp.dot(t0, ax_ref[...],
                               preferred_element_type=jnp.float32)

    @pl.when(i < _S1)
    def _level0():
        x = f0_ref[...].reshape(_BC0 * 192, 192)
        b = jnp.dot(x, cw0_ref[...], preferred_element_type=jnp.float32)
        pooled = jnp.sum(b.reshape(_BC0, 192, _K) * rw0_ref[...][None],
                         axis=1)
        base = i * _BC0
        o_ref[pl.ds(base, _BC0), :] = pooled + pos_embed(base, _BC0)

    @pl.when((i >= _S1) & (i < _S2))
    def _level1():
        @pl.when(i == _S1)
        def _():
            t1 = _up_tent(96, 96, 0, 1)
            rw1_ref[...] = jnp.dot(t1, ay_ref[...],
                                   preferred_element_type=jnp.float32)
            cw1_ref[...] = jnp.dot(t1, ax_ref[...],
                                   preferred_element_type=jnp.float32)
        x = f1_ref[...].reshape(_BC1 * 96, 96)
        b = jnp.dot(x, cw1_ref[...], preferred_element_type=jnp.float32)
        pooled = jnp.sum(b.reshape(_BC1, 96, _K) * rw1_ref[...][None],
                         axis=1)
        base = 192 + (i - _S1) * _BC1
        o_ref[pl.ds(base, _BC1), :] = pooled + pos_embed(base, _BC1)

    @pl.when((i >= _S2) & (i < _S3))
    def _level2():
        @pl.when(i == _S2)
        def _():
            ty = _up_tent(48, 2304, 0, 1, wdiv=48)
            tx = _up_tent_mod(48, 48, 2304)
            w2_ref[...] = (
                jnp.dot(ty, ay_ref[...], preferred_element_type=jnp.float32)
                * jnp.dot(tx, ax_ref[...],
                          preferred_element_type=jnp.float32))
        x = f2_ref[...].reshape(_BC2, 2304)
        pooled = jnp.dot(x, w2_ref[...], preferred_element_type=jnp.float32)
        base = 576 + (i - _S2) * _BC2
        o_ref[pl.ds(base, _BC2), :] = pooled + pos_embed(base, _BC2)

    @pl.when(i >= _S3)
    def _level3():
        @pl.when(i == _S3)
        def _():
            ty = _up_tent(24, 576, 0, 1, wdiv=24)
            tx = _up_tent_mod(24, 24, 576)
            w3_ref[...] = (
                jnp.dot(ty, ay_ref[...], preferred_element_type=jnp.float32)
                * jnp.dot(tx, ax_ref[...],
                          preferred_element_type=jnp.float32))
        x = f3_ref[...].reshape(_BC3, 576)
        pooled = jnp.dot(x, w3_ref[...], preferred_element_type=jnp.float32)
        base = 1344 + (i - _S3) * _BC3
        o_ref[pl.ds(base, _BC3), :] = pooled + pos_embed(base, _BC3)


def kernel(feat0, feat1, feat2, feat3, boxes):
    boxes_t = jnp.transpose(boxes, (1, 0))
    f2 = feat2.reshape(1, 768, 48 * 48)
    f3 = feat3.reshape(1, 1536, 24 * 24)
    full = pl.pallas_call(
        _body,
        out_shape=jax.ShapeDtypeStruct((2880, _K), jnp.float32),
        grid=(_STEPS,),
        in_specs=[
            pl.BlockSpec((4, _K), lambda i: (0, 0)),
            pl.BlockSpec((1, _BC0, 192, 192),
                         lambda i: (0, jnp.minimum(i, _S1 - 1), 0, 0)),
            pl.BlockSpec((1, _BC1, 96, 96),
                         lambda i: (0, jnp.clip(i - _S1, 0, _N1 - 1), 0, 0)),
            pl.BlockSpec((1, _BC2, 2304),
                         lambda i: (0, jnp.clip(i - _S2, 0, _N2 - 1), 0)),
            pl.BlockSpec((1, _BC3, 576),
                         lambda i: (0, jnp.clip(i - _S3, 0, _N3 - 1), 0)),
        ],
        out_specs=pl.BlockSpec((2880, _K), lambda i: (0, 0)),
        scratch_shapes=[
            pltpu.VMEM((192, _K), jnp.float32),
            pltpu.VMEM((192, _K), jnp.float32),
            pltpu.VMEM((192, _K), jnp.float32),
            pltpu.VMEM((192, _K), jnp.float32),
            pltpu.VMEM((96, _K), jnp.float32),
            pltpu.VMEM((96, _K), jnp.float32),
            pltpu.VMEM((2304, _K), jnp.float32),
            pltpu.VMEM((576, _K), jnp.float32),
        ],
        compiler_params=pltpu.CompilerParams(
            dimension_semantics=("arbitrary",)),
        name="roi_fused",
    )(boxes_t, feat0, feat1, f2, f3)
    return jnp.transpose(full, (1, 0))[None]


# speedup vs baseline: 1.5482x; 1.0162x over previous
import math

import jax
import jax.numpy as jnp
from jax.experimental import pallas as pl
from jax.experimental.pallas import tpu as pltpu

_GRID = 192
_SCALE = 0.25
_IMG = 768.0
_OUT = 7
_SR = 2
_NS = _OUT * _SR
_K = 64
_POS_D = 720

_BC0, _BC1, _BC2, _BC3 = 16, 64, 256, 256
_N0, _N1, _N2, _N3 = 192 // _BC0, 384 // _BC1, 768 // _BC2, 1536 // _BC3
_S1 = _N0
_S2 = _S1 + _N1
_S3 = _S2 + _N2
_STEPS = _S3 + _N3


def _up_tent(n, rows, row_iota_dim, col_iota_dim, wdiv=None):
    i = jax.lax.broadcasted_iota(jnp.int32, (rows, _GRID),
                                 col_iota_dim).astype(jnp.float32)
    u = jnp.clip((i + 0.5) * (n / float(_GRID)) - 0.5, 0.0, float(n - 1))
    p = jax.lax.broadcasted_iota(jnp.int32, (rows, _GRID), row_iota_dim)
    if wdiv is not None:
        p = p // wdiv
    r = p.astype(jnp.float32)
    return jnp.maximum(0.0, 1.0 - jnp.abs(u - r))


def _up_tent_mod(n, w, rows):
    i = jax.lax.broadcasted_iota(jnp.int32, (rows, _GRID),
                                 1).astype(jnp.float32)
    u = jnp.clip((i + 0.5) * (n / float(_GRID)) - 0.5, 0.0, float(n - 1))
    p = jax.lax.broadcasted_iota(jnp.int32, (rows, _GRID), 0)
    q = (p - (p // w) * w).astype(jnp.float32)
    return jnp.maximum(0.0, 1.0 - jnp.abs(u - q))


def _body(b_ref, f0_ref, f1_ref, f2_ref, f3_ref, o_ref,
          bt_ref, acc_ref, ay_ref, ax_ref, rw0_ref, cw0_ref, rw1_ref,
          cw1_ref, w2_ref, w3_ref):
    i = pl.program_id(0)

    @pl.when(i == 0)
    def _init_bt():
        bt_ref[...] = jnp.transpose(b_ref[...], (1, 0))

    bT = bt_ref[...]

    def pos_embed(base, bc):
        c = base + jax.lax.broadcasted_iota(jnp.int32, (bc, 1), 0)
        blk = c // _POS_D
        j = c - blk * _POS_D
        expo = (2.0 * (j // 2).astype(jnp.float32)) / float(_POS_D)
        inv_t = jnp.exp(-math.log(10000.0) * expo)
        nx1 = bT[0:1, :] / _IMG
        ny1 = bT[1:2, :] / _IMG
        nw = bT[2:3, :] / _IMG - nx1
        nh = bT[3:4, :] / _IMG - ny1
        v = jnp.where(blk == 0, ny1 + nh * 0.5,
                      jnp.where(blk == 1, nx1 + nw * 0.5,
                                jnp.where(blk == 2, nw, nh)))
        ang = v * (2.0 * math.pi) * inv_t
        return jnp.where(j % 2 == 0, jnp.sin(ang), jnp.cos(ang))

    @pl.when(i == 0)
    def _init():
        x1 = bT[0:1, :] * _SCALE
        y1 = bT[1:2, :] * _SCALE
        x2 = bT[2:3, :] * _SCALE
        y2 = bT[3:4, :] * _SCALE
        roi_w = jnp.maximum(x2 - x1, 1.0)
        roi_h = jnp.maximum(y2 - y1, 1.0)
        g = (jax.lax.broadcasted_iota(jnp.int32, (_NS, _K), 0)
             .astype(jnp.float32) + 0.5) / (2.0 * _OUT)
        xs = x1 + g * roi_w
        ys = y1 + g * roi_h

        def grid_tent(s):
            valid = ((s > -1.0) & (s < float(_GRID))).astype(jnp.float32)
            sc = jnp.clip(s, 0.0, float(_GRID - 1))
            gi = jax.lax.broadcasted_iota(jnp.int32, (_GRID, _NS, _K),
                                          0).astype(jnp.float32)
            t = jnp.maximum(0.0, 1.0 - jnp.abs(sc[None] - gi)) * valid[None]
            return jnp.sum(t, axis=1) * (1.0 / _NS)

        ay_ref[...] = grid_tent(ys)
        ax_ref[...] = grid_tent(xs)
        t0 = _up_tent(192, 192, 0, 1)
        rw0_ref[...] = jnp.dot(t0, ay_ref[...],
                               preferred_element_type=jnp.float32)
        cw0_ref[...] = jnp.dot(t0, ax_ref[...],
                               preferred_element_type=jnp.float32)

    @pl.when(i < _S1)
    def _level0():
        x = f0_ref[...].reshape(_BC0 * 192, 192)
        b = jnp.dot(x, cw0_ref[...], preferred_element_type=jnp.float32)
        pooled = jnp.sum(b.reshape(_BC0, 192, _K) * rw0_ref[...][None],
                         axis=1)
        base = i * _BC0
        acc_ref[pl.ds(base, _BC0), :] = pooled + pos_embed(base, _BC0)

    @pl.when((i >= _S1) & (i < _S2))
    def _level1():
        @pl.when(i == _S1)
        def _():
            t1 = _up_tent(96, 96, 0, 1)
            rw1_ref[...] = jnp.dot(t1, ay_ref[...],
                                   preferred_element_type=jnp.float32)
            cw1_ref[...] = jnp.dot(t1, ax_ref[...],
                                   preferred_element_type=jnp.float32)
        x = f1_ref[...].reshape(_BC1 * 96, 96)
        b = jnp.dot(x, cw1_ref[...], preferred_element_type=jnp.float32)
        pooled = jnp.sum(b.reshape(_BC1, 96, _K) * rw1_ref[...][None],
                         axis=1)
        base = 192 + (i - _S1) * _BC1
        acc_ref[pl.ds(base, _BC1), :] = pooled + pos_embed(base, _BC1)

    @pl.when((i >= _S2) & (i < _S3))
    def _level2():
        @pl.when(i == _S2)
        def _():
            ty = _up_tent(48, 2304, 0, 1, wdiv=48)
            tx = _up_tent_mod(48, 48, 2304)
            w2_ref[...] = (
                jnp.dot(ty, ay_ref[...], preferred_element_type=jnp.float32)
                * jnp.dot(tx, ax_ref[...],
                          preferred_element_type=jnp.float32))
        x = f2_ref[...].reshape(_BC2, 2304)
        pooled = jnp.dot(x, w2_ref[...], preferred_element_type=jnp.float32)
        base = 576 + (i - _S2) * _BC2
        acc_ref[pl.ds(base, _BC2), :] = pooled + pos_embed(base, _BC2)

    @pl.when(i >= _S3)
    def _level3():
        @pl.when(i == _S3)
        def _():
            ty = _up_tent(24, 576, 0, 1, wdiv=24)
            tx = _up_tent_mod(24, 24, 576)
            w3_ref[...] = (
                jnp.dot(ty, ay_ref[...], preferred_element_type=jnp.float32)
                * jnp.dot(tx, ax_ref[...],
                          preferred_element_type=jnp.float32))
        x = f3_ref[...].reshape(_BC3, 576)
        pooled = jnp.dot(x, w3_ref[...], preferred_element_type=jnp.float32)
        base = 1344 + (i - _S3) * _BC3
        acc_ref[pl.ds(base, _BC3), :] = pooled + pos_embed(base, _BC3)

    @pl.when(i == _STEPS - 1)
    def _emit():
        o_ref[...] = jnp.transpose(acc_ref[...], (1, 0))


def kernel(feat0, feat1, feat2, feat3, boxes):
    f2 = feat2.reshape(1, 768, 48 * 48)
    f3 = feat3.reshape(1, 1536, 24 * 24)
    full = pl.pallas_call(
        _body,
        out_shape=jax.ShapeDtypeStruct((_K, 2880), jnp.float32),
        grid=(_STEPS,),
        in_specs=[
            pl.BlockSpec((_K, 4), lambda i: (0, 0)),
            pl.BlockSpec((1, _BC0, 192, 192),
                         lambda i: (0, jnp.minimum(i, _S1 - 1), 0, 0)),
            pl.BlockSpec((1, _BC1, 96, 96),
                         lambda i: (0, jnp.clip(i - _S1, 0, _N1 - 1), 0, 0)),
            pl.BlockSpec((1, _BC2, 2304),
                         lambda i: (0, jnp.clip(i - _S2, 0, _N2 - 1), 0)),
            pl.BlockSpec((1, _BC3, 576),
                         lambda i: (0, jnp.clip(i - _S3, 0, _N3 - 1), 0)),
        ],
        out_specs=pl.BlockSpec((_K, 2880), lambda i: (0, 0)),
        scratch_shapes=[
            pltpu.VMEM((4, _K), jnp.float32),
            pltpu.VMEM((2880, _K), jnp.float32),
            pltpu.VMEM((192, _K), jnp.float32),
            pltpu.VMEM((192, _K), jnp.float32),
            pltpu.VMEM((192, _K), jnp.float32),
            pltpu.VMEM((192, _K), jnp.float32),
            pltpu.VMEM((96, _K), jnp.float32),
            pltpu.VMEM((96, _K), jnp.float32),
            pltpu.VMEM((2304, _K), jnp.float32),
            pltpu.VMEM((576, _K), jnp.float32),
        ],
        compiler_params=pltpu.CompilerParams(
            dimension_semantics=("arbitrary",)),
        name="roi_fused",
    )(boxes, feat0, feat1, f2, f3)
    return full[None]
